# baseline (speedup 1.0000x reference)
import functools

import jax
import jax.numpy as jnp
from jax import lax
from jax.experimental import pallas as pl
from jax.experimental.pallas import tpu as pltpu
from jax.experimental.pallas import tpu_sc as plsc

_GATHER_W = 128
_ROW_BLOCK = 4000


def _project_body(tt_ref, at_ref, wt_ref, bt_ref, wa_ref, ba_ref,
                  pt_ref, pa_ref):
    pt_ref[...] = (jnp.dot(tt_ref[...], wt_ref[...],
                           preferred_element_type=jnp.float32)
                   + bt_ref[...]).astype(jnp.bfloat16)
    pa_ref[...] = (jnp.dot(at_ref[...], wa_ref[...],
                           preferred_element_type=jnp.float32)
                   + ba_ref[...]).astype(jnp.bfloat16)


def _project_tables(time_table, act_table, W_time, b_time, W_act, b_act):
    V, D = time_table.shape
    grid = V // _ROW_BLOCK
    row_spec = pl.BlockSpec((_ROW_BLOCK, D), lambda i: (i, 0))
    full_w = pl.BlockSpec((D, D), lambda i: (0, 0))
    full_b = pl.BlockSpec((1, D), lambda i: (0, 0))
    out_shape = jax.ShapeDtypeStruct((V, D), jnp.bfloat16)
    return pl.pallas_call(
        _project_body,
        grid=(grid,),
        in_specs=[row_spec, row_spec, full_w, full_b, full_w, full_b],
        out_specs=[row_spec, row_spec],
        out_shape=[out_shape, out_shape],
    )(time_table, act_table, W_time, b_time.reshape(1, D),
      W_act, b_act.reshape(1, D))


def _ctx_body(ctx_ref, g_ref, a_ref, o_ref):
    c = ctx_ref[...]
    gv = c[:, 0:1]
    av = c[:, 1:2]
    n = c.shape[0]
    oh_g = (lax.broadcasted_iota(jnp.int32, (n, g_ref.shape[0]), 1)
            == gv).astype(jnp.float32)
    oh_a = (lax.broadcasted_iota(jnp.int32, (n, a_ref.shape[0]), 1)
            == av).astype(jnp.float32)
    g_emb = jnp.dot(oh_g, g_ref[...], preferred_element_type=jnp.float32,
                    precision=lax.Precision.HIGHEST)
    a_emb = jnp.dot(oh_a, a_ref[...], preferred_element_type=jnp.float32,
                    precision=lax.Precision.HIGHEST)
    o_ref[...] = jnp.concatenate([g_emb, a_emb], axis=-1)


def _ctx_embed(context_tokens, gender_table, age_table):
    n = context_tokens.shape[0]
    dg = gender_table.shape[1]
    da = age_table.shape[1]
    return pl.pallas_call(
        _ctx_body,
        out_shape=jax.ShapeDtypeStruct((n, dg + da), jnp.float32),
    )(context_tokens, gender_table, age_table)


def _sc_gather_one(p_tab, idx, step_k):
    V, D = p_tab.shape
    n_rows, W = idx.shape
    n_idx = n_rows * W
    step_rows = step_k * W
    mesh = plsc.VectorSubcoreMesh(core_axis_name="c", subcore_axis_name="s")
    out_t = jax.ShapeDtypeStruct((n_idx, D), jnp.bfloat16)

    @functools.partial(
        pl.kernel, mesh=mesh, out_type=out_t,
        scratch_types=[pltpu.SemaphoreType.DMA],
        compiler_params=pltpu.CompilerParams(use_tc_tiling_on_sc=False))
    def k(p_hbm, i_hbm, o_hbm, sem):
        def body(i_v, o_v):
            copies = []
            for j in range(step_k):
                copies.append(pltpu.async_copy(
                    p_hbm.at[i_v.at[j]], o_v.at[pl.ds(j * W, W)], sem))
            for c in copies:
                c.wait()

        pltpu.emit_pipeline(
            body,
            grid=(n_rows // step_k,),
            in_specs=[pl.BlockSpec((step_k, W), lambda i: (i, 0))],
            out_specs=[pl.BlockSpec((step_rows, D), lambda i: (i, 0))],
            core_axis_name=("c", "s"),
            dimension_semantics=(pltpu.PARALLEL,),
        )(i_hbm, o_hbm)

    return k(p_tab, idx)


def kernel(context_tokens, time_tokens, act_tokens, time_table, act_table,
           age_table, gender_table, W_time, b_time, W_act, b_act):
    B, L = time_tokens.shape
    D = time_table.shape[1]
    t_idx = time_tokens.astype(jnp.int32).reshape(B * L // _GATHER_W, _GATHER_W)
    a_idx = act_tokens.astype(jnp.int32).reshape(B * L // _GATHER_W, _GATHER_W)

    p_time, p_act = _project_tables(time_table, act_table,
                                    W_time, b_time, W_act, b_act)
    ctx_emb = _ctx_embed(context_tokens.astype(jnp.int32),
                         gender_table, age_table)
    t_flat = _sc_gather_one(p_time, t_idx, 2)
    a_flat = _sc_gather_one(p_act, a_idx, 2)
    return (ctx_emb,
            t_flat.reshape(B, L, D).astype(jnp.float32),
            a_flat.reshape(B, L, D).astype(jnp.float32))


# speedup vs baseline: 1.5198x; 1.5198x over previous
import functools

import jax
import jax.numpy as jnp
from jax import lax
from jax.experimental import pallas as pl
from jax.experimental.pallas import tpu as pltpu
from jax.experimental.pallas import tpu_sc as plsc

_GATHER_W = 128
_ROW_BLOCK = 4000


def _project_body(tt_ref, at_ref, wt_ref, bt_ref, wa_ref, ba_ref,
                  pt_ref, pa_ref):
    pt_ref[...] = jnp.dot(tt_ref[...], wt_ref[...],
                          preferred_element_type=jnp.float32) + bt_ref[...]
    pa_ref[...] = jnp.dot(at_ref[...], wa_ref[...],
                          preferred_element_type=jnp.float32) + ba_ref[...]


def _project_tables(time_table, act_table, W_time, b_time, W_act, b_act):
    V, D = time_table.shape
    grid = V // _ROW_BLOCK
    row_spec = pl.BlockSpec((_ROW_BLOCK, D), lambda i: (i, 0))
    full_w = pl.BlockSpec((D, D), lambda i: (0, 0))
    full_b = pl.BlockSpec((1, D), lambda i: (0, 0))
    out_shape = jax.ShapeDtypeStruct((V, D), jnp.float32)
    return pl.pallas_call(
        _project_body,
        grid=(grid,),
        in_specs=[row_spec, row_spec, full_w, full_b, full_w, full_b],
        out_specs=[row_spec, row_spec],
        out_shape=[out_shape, out_shape],
    )(time_table, act_table, W_time, b_time.reshape(1, D),
      W_act, b_act.reshape(1, D))


def _ctx_body(ctx_ref, g_ref, a_ref, o_ref):
    c = ctx_ref[...]
    gv = c[:, 0:1]
    av = c[:, 1:2]
    n = c.shape[0]
    oh_g = (lax.broadcasted_iota(jnp.int32, (n, g_ref.shape[0]), 1)
            == gv).astype(jnp.float32)
    oh_a = (lax.broadcasted_iota(jnp.int32, (n, a_ref.shape[0]), 1)
            == av).astype(jnp.float32)
    g_emb = jnp.dot(oh_g, g_ref[...], preferred_element_type=jnp.float32,
                    precision=lax.Precision.HIGHEST)
    a_emb = jnp.dot(oh_a, a_ref[...], preferred_element_type=jnp.float32,
                    precision=lax.Precision.HIGHEST)
    o_ref[...] = jnp.concatenate([g_emb, a_emb], axis=-1)


def _ctx_embed(context_tokens, gender_table, age_table):
    n = context_tokens.shape[0]
    dg = gender_table.shape[1]
    da = age_table.shape[1]
    return pl.pallas_call(
        _ctx_body,
        out_shape=jax.ShapeDtypeStruct((n, dg + da), jnp.float32),
    )(context_tokens, gender_table, age_table)


def _sc_gather_one(p_tab, idx, step_k):
    V, D = p_tab.shape
    n_rows, W = idx.shape
    n_idx = n_rows * W
    step_rows = step_k * W
    mesh = plsc.VectorSubcoreMesh(core_axis_name="c", subcore_axis_name="s")
    out_t = jax.ShapeDtypeStruct((n_idx, D), jnp.float32)

    @functools.partial(
        pl.kernel, mesh=mesh, out_type=out_t,
        scratch_types=[pltpu.SemaphoreType.DMA],
        compiler_params=pltpu.CompilerParams(use_tc_tiling_on_sc=False))
    def k(p_hbm, i_hbm, o_hbm, sem):
        def body(i_v, o_v):
            copies = []
            for j in range(step_k):
                copies.append(pltpu.async_copy(
                    p_hbm.at[i_v.at[j]], o_v.at[pl.ds(j * W, W)], sem))
            for c in copies:
                c.wait()

        pltpu.emit_pipeline(
            body,
            grid=(n_rows // step_k,),
            in_specs=[pl.BlockSpec((step_k, W), lambda i: (i, 0))],
            out_specs=[pl.BlockSpec((step_rows, D), lambda i: (i, 0))],
            core_axis_name=("c", "s"),
            dimension_semantics=(pltpu.PARALLEL,),
        )(i_hbm, o_hbm)

    return k(p_tab, idx)


def kernel(context_tokens, time_tokens, act_tokens, time_table, act_table,
           age_table, gender_table, W_time, b_time, W_act, b_act):
    B, L = time_tokens.shape
    D = time_table.shape[1]
    t_idx = time_tokens.astype(jnp.int32).reshape(B * L // _GATHER_W, _GATHER_W)
    a_idx = act_tokens.astype(jnp.int32).reshape(B * L // _GATHER_W, _GATHER_W)

    p_time, p_act = _project_tables(time_table, act_table,
                                    W_time, b_time, W_act, b_act)
    ctx_emb = _ctx_embed(context_tokens.astype(jnp.int32),
                         gender_table, age_table)
    t_flat = _sc_gather_one(p_time, t_idx, 2)
    a_flat = _sc_gather_one(p_act, a_idx, 2)
    return ctx_emb, t_flat.reshape(B, L, D), a_flat.reshape(B, L, D)


# speedup vs baseline: 1.5454x; 1.0169x over previous
import functools

import jax
import jax.numpy as jnp
from jax import lax
from jax.experimental import pallas as pl
from jax.experimental.pallas import tpu as pltpu
from jax.experimental.pallas import tpu_sc as plsc

_GATHER_W = 128
_ROW_BLOCK = 4000


def _project_body(tt_ref, at_ref, wt_ref, bt_ref, wa_ref, ba_ref, p_ref):
    D = tt_ref.shape[1]
    p_ref[:, :D] = jnp.dot(tt_ref[...], wt_ref[...],
                           preferred_element_type=jnp.float32) + bt_ref[...]
    p_ref[:, D:] = jnp.dot(at_ref[...], wa_ref[...],
                           preferred_element_type=jnp.float32) + ba_ref[...]


def _project_tables(time_table, act_table, W_time, b_time, W_act, b_act):
    V, D = time_table.shape
    grid = V // _ROW_BLOCK
    row_spec = pl.BlockSpec((_ROW_BLOCK, D), lambda i: (i, 0))
    out_spec = pl.BlockSpec((_ROW_BLOCK, 2 * D), lambda i: (i, 0))
    full_w = pl.BlockSpec((D, D), lambda i: (0, 0))
    full_b = pl.BlockSpec((1, D), lambda i: (0, 0))
    out_shape = jax.ShapeDtypeStruct((V, 2 * D), jnp.float32)
    return pl.pallas_call(
        _project_body,
        grid=(grid,),
        in_specs=[row_spec, row_spec, full_w, full_b, full_w, full_b],
        out_specs=out_spec,
        out_shape=out_shape,
    )(time_table, act_table, W_time, b_time.reshape(1, D),
      W_act, b_act.reshape(1, D))


def _ctx_body(ctx_ref, g_ref, a_ref, o_ref):
    c = ctx_ref[...]
    gv = c[:, 0:1]
    av = c[:, 1:2]
    n = c.shape[0]
    oh_g = (lax.broadcasted_iota(jnp.int32, (n, g_ref.shape[0]), 1)
            == gv).astype(jnp.float32)
    oh_a = (lax.broadcasted_iota(jnp.int32, (n, a_ref.shape[0]), 1)
            == av).astype(jnp.float32)
    g_emb = jnp.dot(oh_g, g_ref[...], preferred_element_type=jnp.float32,
                    precision=lax.Precision.HIGHEST)
    a_emb = jnp.dot(oh_a, a_ref[...], preferred_element_type=jnp.float32,
                    precision=lax.Precision.HIGHEST)
    o_ref[...] = jnp.concatenate([g_emb, a_emb], axis=-1)


def _ctx_embed(context_tokens, gender_table, age_table):
    n = context_tokens.shape[0]
    dg = gender_table.shape[1]
    da = age_table.shape[1]
    return pl.pallas_call(
        _ctx_body,
        out_shape=jax.ShapeDtypeStruct((n, dg + da), jnp.float32),
    )(context_tokens, gender_table, age_table)


def _sc_gather_one(p_comb, idx, step_k):
    V, D2 = p_comb.shape
    n_rows, W = idx.shape
    n_idx = n_rows * W
    step_rows = step_k * W
    mesh = plsc.VectorSubcoreMesh(core_axis_name="c", subcore_axis_name="s")
    out_t = jax.ShapeDtypeStruct((n_idx, D2), jnp.float32)

    @functools.partial(
        pl.kernel, mesh=mesh, out_type=out_t,
        scratch_types=[pltpu.SemaphoreType.DMA],
        compiler_params=pltpu.CompilerParams(use_tc_tiling_on_sc=True))
    def k(p_hbm, i_hbm, o_hbm, sem):
        def body(i_v, o_v):
            copies = []
            for j in range(step_k):
                copies.append(pltpu.async_copy(
                    p_hbm.at[i_v.at[j]], o_v.at[pl.ds(j * W, W)], sem))
            for c in copies:
                c.wait()

        pltpu.emit_pipeline(
            body,
            grid=(n_rows // step_k,),
            in_specs=[pl.BlockSpec((step_k, W), lambda i: (i, 0))],
            out_specs=[pl.BlockSpec((step_rows, D2), lambda i: (i, 0))],
            core_axis_name=("c", "s"),
            dimension_semantics=(pltpu.PARALLEL,),
        )(i_hbm, o_hbm)

    return k(p_comb, idx)


def kernel(context_tokens, time_tokens, act_tokens, time_table, act_table,
           age_table, gender_table, W_time, b_time, W_act, b_act):
    B, L = time_tokens.shape
    D = time_table.shape[1]
    t_idx = time_tokens.astype(jnp.int32).reshape(B * L // _GATHER_W, _GATHER_W)
    a_idx = act_tokens.astype(jnp.int32).reshape(B * L // _GATHER_W, _GATHER_W)

    p_comb = _project_tables(time_table, act_table,
                             W_time, b_time, W_act, b_act)
    ctx_emb = _ctx_embed(context_tokens.astype(jnp.int32),
                         gender_table, age_table)
    t_wide = _sc_gather_one(p_comb, t_idx, 2)
    a_wide = _sc_gather_one(p_comb, a_idx, 2)
    t_emb = t_wide[:, :D].reshape(B, L, D)
    a_emb = a_wide[:, D:].reshape(B, L, D)
    return ctx_emb, t_emb, a_emb


# speedup vs baseline: 1.5852x; 1.0257x over previous
import functools

import jax
import jax.numpy as jnp
from jax import lax
from jax.experimental import pallas as pl
from jax.experimental.pallas import tpu as pltpu
from jax.experimental.pallas import tpu_sc as plsc

_GATHER_W = 128
_ROW_BLOCK = 4000


def _project_body(tt_ref, at_ref, wt_ref, bt_ref, wa_ref, ba_ref, p_ref):
    D = tt_ref.shape[1]
    p_ref[:, :D] = jnp.dot(tt_ref[...], wt_ref[...],
                           preferred_element_type=jnp.float32) + bt_ref[...]
    p_ref[:, D:] = jnp.dot(at_ref[...], wa_ref[...],
                           preferred_element_type=jnp.float32) + ba_ref[...]


def _project_tables(time_table, act_table, W_time, b_time, W_act, b_act):
    V, D = time_table.shape
    grid = V // _ROW_BLOCK
    row_spec = pl.BlockSpec((_ROW_BLOCK, D), lambda i: (i, 0))
    out_spec = pl.BlockSpec((_ROW_BLOCK, 2 * D), lambda i: (i, 0))
    full_w = pl.BlockSpec((D, D), lambda i: (0, 0))
    full_b = pl.BlockSpec((1, D), lambda i: (0, 0))
    out_shape = jax.ShapeDtypeStruct((V, 2 * D), jnp.float32)
    return pl.pallas_call(
        _project_body,
        grid=(grid,),
        in_specs=[row_spec, row_spec, full_w, full_b, full_w, full_b],
        out_specs=out_spec,
        out_shape=out_shape,
    )(time_table, act_table, W_time, b_time.reshape(1, D),
      W_act, b_act.reshape(1, D))


def _ctx_body(ctx_ref, g_ref, a_ref, o_ref):
    c = ctx_ref[...]
    gv = c[:, 0:1]
    av = c[:, 1:2]
    n = c.shape[0]
    oh_g = (lax.broadcasted_iota(jnp.int32, (n, g_ref.shape[0]), 1)
            == gv).astype(jnp.float32)
    oh_a = (lax.broadcasted_iota(jnp.int32, (n, a_ref.shape[0]), 1)
            == av).astype(jnp.float32)
    g_emb = jnp.dot(oh_g, g_ref[...], preferred_element_type=jnp.float32,
                    precision=lax.Precision.HIGHEST)
    a_emb = jnp.dot(oh_a, a_ref[...], preferred_element_type=jnp.float32,
                    precision=lax.Precision.HIGHEST)
    o_ref[...] = jnp.concatenate([g_emb, a_emb], axis=-1)


def _ctx_embed(context_tokens, gender_table, age_table):
    n = context_tokens.shape[0]
    dg = gender_table.shape[1]
    da = age_table.shape[1]
    return pl.pallas_call(
        _ctx_body,
        out_shape=jax.ShapeDtypeStruct((n, dg + da), jnp.float32),
    )(context_tokens, gender_table, age_table)


def _sc_gather_one(p_comb, idx, step_k):
    V, D2 = p_comb.shape
    n_rows, W = idx.shape
    n_idx = n_rows * W
    step_rows = step_k * W
    mesh = plsc.VectorSubcoreMesh(core_axis_name="c", subcore_axis_name="s")
    out_t = jax.ShapeDtypeStruct((n_idx, D2), jnp.float32)

    @functools.partial(
        pl.kernel, mesh=mesh, out_type=out_t,
        scratch_types=[pltpu.SemaphoreType.DMA],
        compiler_params=pltpu.CompilerParams(use_tc_tiling_on_sc=True))
    def k(p_hbm, i_hbm, o_hbm, sem):
        def body(i_v, o_v):
            copies = []
            for j in range(step_k):
                copies.append(pltpu.async_copy(
                    p_hbm.at[i_v.at[j]], o_v.at[pl.ds(j * W, W)], sem))
            for c in copies:
                c.wait()

        pltpu.emit_pipeline(
            body,
            grid=(n_rows // step_k,),
            in_specs=[pl.BlockSpec((step_k, W), lambda i: (i, 0))],
            out_specs=[pl.BlockSpec((step_rows, D2), lambda i: (i, 0))],
            core_axis_name=("c", "s"),
            dimension_semantics=(pltpu.PARALLEL,),
        )(i_hbm, o_hbm)

    return k(p_comb, idx)


def kernel(context_tokens, time_tokens, act_tokens, time_table, act_table,
           age_table, gender_table, W_time, b_time, W_act, b_act):
    B, L = time_tokens.shape
    D = time_table.shape[1]
    t_idx = time_tokens.astype(jnp.int32).reshape(B * L // _GATHER_W, _GATHER_W)
    a_idx = act_tokens.astype(jnp.int32).reshape(B * L // _GATHER_W, _GATHER_W)

    p_comb = _project_tables(time_table, act_table,
                             W_time, b_time, W_act, b_act)
    ctx_emb = _ctx_embed(context_tokens.astype(jnp.int32),
                         gender_table, age_table)
    t_wide = _sc_gather_one(p_comb, t_idx, 2)
    a_wide = _sc_gather_one(p_comb, a_idx, 2)
    t_wide, a_wide = lax.optimization_barrier((t_wide, a_wide))
    t_emb = t_wide[:, :D].reshape(B, L, D)
    a_emb = a_wide[:, D:].reshape(B, L, D)
    return ctx_emb, t_emb, a_emb


# speedup vs baseline: 1.8739x; 1.1822x over previous
import functools

import jax
import jax.numpy as jnp
from jax import lax
from jax.experimental import pallas as pl
from jax.experimental.pallas import tpu as pltpu
from jax.experimental.pallas import tpu_sc as plsc

_GATHER_W = 128
_ROW_BLOCK = 4000


def _project_body(tt_ref, at_ref, wt_ref, bt_ref, wa_ref, ba_ref,
                  pta_ref, pat_ref):
    D = tt_ref.shape[1]
    t_proj = jnp.dot(tt_ref[...], wt_ref[...],
                     preferred_element_type=jnp.float32) + bt_ref[...]
    a_proj = jnp.dot(at_ref[...], wa_ref[...],
                     preferred_element_type=jnp.float32) + ba_ref[...]
    pta_ref[:, :D] = t_proj
    pta_ref[:, D:] = a_proj
    pat_ref[:, :D] = a_proj
    pat_ref[:, D:] = t_proj


def _project_tables(time_table, act_table, W_time, b_time, W_act, b_act):
    V, D = time_table.shape
    grid = V // _ROW_BLOCK
    row_spec = pl.BlockSpec((_ROW_BLOCK, D), lambda i: (i, 0))
    out_spec = pl.BlockSpec((_ROW_BLOCK, 2 * D), lambda i: (i, 0))
    full_w = pl.BlockSpec((D, D), lambda i: (0, 0))
    full_b = pl.BlockSpec((1, D), lambda i: (0, 0))
    out_shape = jax.ShapeDtypeStruct((V, 2 * D), jnp.float32)
    return pl.pallas_call(
        _project_body,
        grid=(grid,),
        in_specs=[row_spec, row_spec, full_w, full_b, full_w, full_b],
        out_specs=[out_spec, out_spec],
        out_shape=[out_shape, out_shape],
    )(time_table, act_table, W_time, b_time.reshape(1, D),
      W_act, b_act.reshape(1, D))


def _ctx_body(ctx_ref, g_ref, a_ref, o_ref):
    c = ctx_ref[...]
    gv = c[:, 0:1]
    av = c[:, 1:2]
    n = c.shape[0]
    oh_g = (lax.broadcasted_iota(jnp.int32, (n, g_ref.shape[0]), 1)
            == gv).astype(jnp.float32)
    oh_a = (lax.broadcasted_iota(jnp.int32, (n, a_ref.shape[0]), 1)
            == av).astype(jnp.float32)
    g_emb = jnp.dot(oh_g, g_ref[...], preferred_element_type=jnp.float32,
                    precision=lax.Precision.HIGHEST)
    a_emb = jnp.dot(oh_a, a_ref[...], preferred_element_type=jnp.float32,
                    precision=lax.Precision.HIGHEST)
    o_ref[...] = jnp.concatenate([g_emb, a_emb], axis=-1)


def _ctx_embed(context_tokens, gender_table, age_table):
    n = context_tokens.shape[0]
    dg = gender_table.shape[1]
    da = age_table.shape[1]
    return pl.pallas_call(
        _ctx_body,
        out_shape=jax.ShapeDtypeStruct((n, dg + da), jnp.float32),
    )(context_tokens, gender_table, age_table)


def _sc_gather_one(p_comb, idx, step_k):
    V, D2 = p_comb.shape
    n_rows, W = idx.shape
    n_idx = n_rows * W
    step_rows = step_k * W
    mesh = plsc.VectorSubcoreMesh(core_axis_name="c", subcore_axis_name="s")
    out_t = jax.ShapeDtypeStruct((n_idx, D2), jnp.float32)

    @functools.partial(
        pl.kernel, mesh=mesh, out_type=out_t,
        scratch_types=[pltpu.SemaphoreType.DMA],
        compiler_params=pltpu.CompilerParams(use_tc_tiling_on_sc=True))
    def k(p_hbm, i_hbm, o_hbm, sem):
        def body(i_v, o_v):
            copies = []
            for j in range(step_k):
                copies.append(pltpu.async_copy(
                    p_hbm.at[i_v.at[j]], o_v.at[pl.ds(j * W, W)], sem))
            for c in copies:
                c.wait()

        pltpu.emit_pipeline(
            body,
            grid=(n_rows // step_k,),
            in_specs=[pl.BlockSpec((step_k, W), lambda i: (i, 0))],
            out_specs=[pl.BlockSpec((step_rows, D2), lambda i: (i, 0))],
            core_axis_name=("c", "s"),
            dimension_semantics=(pltpu.PARALLEL,),
        )(i_hbm, o_hbm)

    return k(p_comb, idx)


def kernel(context_tokens, time_tokens, act_tokens, time_table, act_table,
           age_table, gender_table, W_time, b_time, W_act, b_act):
    B, L = time_tokens.shape
    D = time_table.shape[1]
    t_idx = time_tokens.astype(jnp.int32).reshape(B * L // _GATHER_W, _GATHER_W)
    a_idx = act_tokens.astype(jnp.int32).reshape(B * L // _GATHER_W, _GATHER_W)

    p_ta, p_at = _project_tables(time_table, act_table,
                                 W_time, b_time, W_act, b_act)
    ctx_emb = _ctx_embed(context_tokens.astype(jnp.int32),
                         gender_table, age_table)
    t_wide = _sc_gather_one(p_ta, t_idx, 2)
    a_wide = _sc_gather_one(p_at, a_idx, 2)
    t_wide, a_wide = lax.optimization_barrier((t_wide, a_wide))
    t_emb = t_wide[:, :D].reshape(B, L, D)
    a_emb = a_wide[:, :D].reshape(B, L, D)
    return ctx_emb, t_emb, a_emb
